# Initial kernel scaffold; baseline (speedup 1.0000x reference)
import jax
import jax.numpy as jnp
from jax.experimental import pallas as pl


def kernel(x, edge_index, batch, atom_emb, W_lin, b_lin, W1, b1, W2, b2):
    raise NotImplementedError("write your pallas kernel here")


# speedup vs baseline: 10.0312x; 10.0312x over previous
import functools

import jax
import jax.numpy as jnp
from jax import lax
from jax.experimental import pallas as pl
from jax.experimental.pallas import tpu as pltpu
from jax.experimental.pallas import tpu_sc as plsc

N = 10000
E = 320000
DIM = 128
EMB = 64
FIXED = 34
NTYPES = 44

NC = 2
NS = 16
L = 16
NW = NC * NS

EB = 128
EPW = ((E + NW * EB - 1) // (NW * EB)) * EB
NB = EPW // EB
EPAD = NW * EPW
NPAD = 10240
RPW = NPAD // NS
CPW = 632
NHP = NS * CPW

RB = 1000
NRB = N // RB


@functools.cache
def _build_sc_degree():
    mesh = plsc.VectorSubcoreMesh(core_axis_name="c", subcore_axis_name="s")
    return functools.partial(
        pl.kernel, mesh=mesh,
        out_type=jax.ShapeDtypeStruct((NC, NHP, L), jnp.float32),
        scratch_types=[
            pltpu.VMEM((EB,), jnp.int32),
            pltpu.VMEM((EB, L), jnp.float32),
            pltpu.VMEM_SHARED((NPAD, L), jnp.float32),
        ],
    )(_sc_degree_body)


def _sc_degree(dst_r):
    return _build_sc_degree()(dst_r)


def _sc_degree_body(dst_hbm, out_hbm, idx_v, buf_v, acc_sh):
    c = lax.axis_index("c")
    s = lax.axis_index("s")
    w = c * NS + s

    @pl.loop(0, EB)
    def _(r):
        buf_v[r, :] = jnp.zeros((L,), jnp.float32)

    @pl.loop(0, RPW // EB)
    def _(j):
        pltpu.sync_copy(buf_v, acc_sh.at[pl.ds(s * RPW + j * EB, EB)])

    plsc.subcore_barrier()

    @pl.loop(0, EB)
    def _(r):
        buf_v[r, :] = jnp.ones((L,), jnp.float32)

    @pl.loop(0, NB)
    def _(b):
        off = pl.multiple_of(w * EPW + b * EB, EB)
        pltpu.sync_copy(dst_hbm.at[pl.ds(off, EB)], idx_v)
        pltpu.sync_copy(buf_v, acc_sh.at[idx_v], add=True)

    plsc.subcore_barrier()
    row = pl.multiple_of(s * CPW, 8)
    pltpu.sync_copy(acc_sh.at[pl.ds(row, CPW)],
                    out_hbm.at[c, pl.ds(row, CPW)])


@functools.cache
def _build_sc_aggregate():
    mesh = plsc.VectorSubcoreMesh(core_axis_name="c", subcore_axis_name="s")
    return functools.partial(
        pl.kernel, mesh=mesh,
        out_type=jax.ShapeDtypeStruct((NC, NHP, DIM), jnp.float32),
        scratch_types=[
            pltpu.VMEM((EB,), jnp.int32),
            pltpu.VMEM((EB,), jnp.int32),
            pltpu.VMEM((EB, DIM), jnp.float32),
            pltpu.VMEM_SHARED((NPAD, DIM), jnp.float32),
        ],
    )(_sc_aggregate_body)


def _sc_aggregate(hws, src_r, dst_r):
    return _build_sc_aggregate()(hws, src_r, dst_r)


def _sc_aggregate_body(hws_hbm, src_hbm, dst_hbm, out_hbm, si_v, di_v, rows_v, acc_sh):
    c = lax.axis_index("c")
    s = lax.axis_index("s")
    w = c * NS + s

    @pl.loop(0, EB)
    def _(r):
        @pl.loop(0, DIM // L)
        def _(j):
            rows_v[r, pl.ds(j * L, L)] = jnp.zeros((L,), jnp.float32)

    @pl.loop(0, RPW // EB)
    def _(j):
        pltpu.sync_copy(rows_v, acc_sh.at[pl.ds(s * RPW + j * EB, EB)])

    plsc.subcore_barrier()

    @pl.loop(0, NB)
    def _(b):
        off = pl.multiple_of(w * EPW + b * EB, EB)
        pltpu.sync_copy(src_hbm.at[pl.ds(off, EB)], si_v)
        pltpu.sync_copy(dst_hbm.at[pl.ds(off, EB)], di_v)
        pltpu.sync_copy(hws_hbm.at[si_v], rows_v)
        pltpu.sync_copy(rows_v, acc_sh.at[di_v], add=True)

    plsc.subcore_barrier()
    row = pl.multiple_of(s * CPW, 8)
    pltpu.sync_copy(acc_sh.at[pl.ds(row, CPW)],
                    out_hbm.at[c, pl.ds(row, CPW)])


def _tc_front_body(x_ref, emb_ref, wl_ref, bl_ref, hist_ref, w1_ref,
                   hws_ref, dinv_ref):
    xb = x_ref[...]
    xt = xb[:, :NTYPES]
    m = jnp.max(xt, axis=1, keepdims=True)
    iota = lax.broadcasted_iota(jnp.int32, xt.shape, 1)
    idx = jnp.min(jnp.where(xt == m, iota, NTYPES), axis=1, keepdims=True)
    onehot = (iota == idx).astype(jnp.float32)
    table = jnp.dot(emb_ref[...], wl_ref[:EMB, :],
                    preferred_element_type=jnp.float32)
    h = jnp.dot(onehot, table, preferred_element_type=jnp.float32)
    h = h + jnp.dot(xb[:, NTYPES:], wl_ref[EMB:, :],
                    preferred_element_type=jnp.float32)
    h = jnp.maximum(h + bl_ref[...], 0.0)
    hw = jnp.dot(h, w1_ref[...], preferred_element_type=jnp.float32)
    deg = hist_ref[0, :, :1] + hist_ref[1, :, :1] + 1.0
    dinv = lax.rsqrt(deg)
    hws_ref[...] = hw * dinv
    dinv_ref[...] = dinv


def _tc_front(x, atom_emb, W_lin, b_lin, hist, W1):
    return pl.pallas_call(
        _tc_front_body,
        grid=(NRB,),
        in_specs=[
            pl.BlockSpec((RB, NTYPES + FIXED), lambda i: (i, 0)),
            pl.BlockSpec((NTYPES, EMB), lambda i: (0, 0)),
            pl.BlockSpec((EMB + FIXED, DIM), lambda i: (0, 0)),
            pl.BlockSpec((1, DIM), lambda i: (0, 0)),
            pl.BlockSpec((NC, RB, L), lambda i: (0, i, 0)),
            pl.BlockSpec((DIM, DIM), lambda i: (0, 0)),
        ],
        out_specs=[
            pl.BlockSpec((RB, DIM), lambda i: (i, 0)),
            pl.BlockSpec((RB, 1), lambda i: (i, 0)),
        ],
        out_shape=[
            jax.ShapeDtypeStruct((N, DIM), jnp.float32),
            jax.ShapeDtypeStruct((N, 1), jnp.float32),
        ],
    )(x, atom_emb, W_lin, b_lin.reshape(1, DIM), hist, W1)


def _tc_mid_body(a_ref, hws_ref, dinv_ref, b_ref, w_ref, out_ref):
    dinv = dinv_ref[...]
    h = dinv * (a_ref[0] + a_ref[1] + hws_ref[...]) + b_ref[...]
    h = jnp.maximum(h, 0.0)
    out_ref[...] = jnp.dot(h, w_ref[...], preferred_element_type=jnp.float32) * dinv


def _tc_mid(acc, hws, dinv, b, W):
    return pl.pallas_call(
        _tc_mid_body,
        grid=(NRB,),
        in_specs=[
            pl.BlockSpec((NC, RB, DIM), lambda i: (0, i, 0)),
            pl.BlockSpec((RB, DIM), lambda i: (i, 0)),
            pl.BlockSpec((RB, 1), lambda i: (i, 0)),
            pl.BlockSpec((1, DIM), lambda i: (0, 0)),
            pl.BlockSpec((DIM, DIM), lambda i: (0, 0)),
        ],
        out_specs=pl.BlockSpec((RB, DIM), lambda i: (i, 0)),
        out_shape=jax.ShapeDtypeStruct((N, DIM), jnp.float32),
    )(acc, hws, dinv, b.reshape(1, DIM), W)


def _tc_final_body(a_ref, hws_ref, dinv_ref, b_ref, out_ref):
    h = dinv_ref[...] * (a_ref[0] + a_ref[1] + hws_ref[...]) + b_ref[...]
    out_ref[...] = jnp.maximum(h, 0.0)


def _tc_final(acc, hws, dinv, b):
    return pl.pallas_call(
        _tc_final_body,
        grid=(NRB,),
        in_specs=[
            pl.BlockSpec((NC, RB, DIM), lambda i: (0, i, 0)),
            pl.BlockSpec((RB, DIM), lambda i: (i, 0)),
            pl.BlockSpec((RB, 1), lambda i: (i, 0)),
            pl.BlockSpec((1, DIM), lambda i: (0, 0)),
        ],
        out_specs=pl.BlockSpec((RB, DIM), lambda i: (i, 0)),
        out_shape=jax.ShapeDtypeStruct((N, DIM), jnp.float32),
    )(acc, hws, dinv, b.reshape(1, DIM))


def kernel(x, edge_index, batch, atom_emb, W_lin, b_lin, W1, b1, W2, b2):
    del batch
    pad = EPAD - E
    src_r = jnp.concatenate([edge_index[0], jnp.zeros((pad,), jnp.int32)])
    dst_r = jnp.concatenate([edge_index[1], jnp.full((pad,), N, jnp.int32)])

    hist = _sc_degree(dst_r)
    hws1, dinv = _tc_front(x, atom_emb, W_lin, b_lin, hist, W1)
    acc1 = _sc_aggregate(hws1, src_r, dst_r)
    hws2 = _tc_mid(acc1, hws1, dinv, b1, W2)
    acc2 = _sc_aggregate(hws2, src_r, dst_r)
    return _tc_final(acc2, hws2, dinv, b2)
